# Initial kernel scaffold; baseline (speedup 1.0000x reference)
import jax
import jax.numpy as jnp
from jax.experimental import pallas as pl


def kernel(query_x, node_x, node_f, W1, b1, W2, b2, W_src, b_src, W_sh, W_gate, a_attn, W_out, edge_src, edge_dst):
    raise NotImplementedError("write your pallas kernel here")


# speedup vs baseline: 9.5743x; 9.5743x over previous
import functools

import jax
import jax.numpy as jnp
from jax import lax
from jax.experimental import pallas as pl
from jax.experimental.pallas import tpu as pltpu
from jax.experimental.pallas import tpu_sc as plsc

_NQ = 10000
_NP = 10000
_E = 160000
_D = 256
_H = 8
_DH = 32
_LE = 64
_CUT = 10.0

_NW = 32
_CH = 128
_EP = 163840
_EW = _EP // _NW
_NCH = _EW // _CH
_PR = 2
_PROWS = _NQ // _PR
_PSR = 5120
_WR = 125

def _mesh():
    return plsc.VectorSubcoreMesh(core_axis_name="c", subcore_axis_name="s")


def _node_mm_body(x_ref, w_ref, b_ref, o_ref):
    o_ref[...] = (
        jnp.dot(x_ref[...], w_ref[...], preferred_element_type=jnp.float32)
        + b_ref[...]
    )


def _node_transform(node_f, W_src, b_src):
    blk = 2000
    return pl.pallas_call(
        _node_mm_body,
        grid=(_NP // blk,),
        in_specs=[
            pl.BlockSpec((blk, _D), lambda i: (i, 0)),
            pl.BlockSpec((_D, _D), lambda i: (0, 0)),
            pl.BlockSpec((_D,), lambda i: (0,)),
        ],
        out_specs=pl.BlockSpec((blk, _D), lambda i: (i, 0)),
        out_shape=jax.ShapeDtypeStruct((_NP, _D), jnp.float32),
    )(node_f, W_src, b_src)


def _gather(g, nxp, qxp, esrc, edst):
    @functools.partial(
        pl.kernel,
        mesh=_mesh(),
        out_type=[
            jax.ShapeDtypeStruct((_EP, _D), jnp.float32),
            jax.ShapeDtypeStruct((_EP, 16), jnp.float32),
            jax.ShapeDtypeStruct((_EP, 16), jnp.float32),
        ],
        scratch_types=[
            pltpu.VMEM((_CH,), jnp.int32),
            pltpu.VMEM((_CH,), jnp.int32),
            pltpu.VMEM((_CH, _D), jnp.float32),
            pltpu.VMEM((_CH, 16), jnp.float32),
            pltpu.VMEM((_CH, 16), jnp.float32),
            pltpu.SemaphoreType.DMA,
        ],
        compiler_params=pltpu.CompilerParams(use_tc_tiling_on_sc=False, needs_layout_passes=False),
    )
    def k(g_hbm, nxp_hbm, qxp_hbm, esrc_hbm, edst_hbm,
          ge_hbm, pxg_hbm, qxg_hbm, si, di, gb, pb, qb, sem):
        wid = lax.axis_index("s") * 2 + lax.axis_index("c")

        def body(j, carry):
            base = wid * _EW + j * _CH
            pltpu.sync_copy(esrc_hbm.at[pl.ds(base, _CH)], si)
            pltpu.sync_copy(edst_hbm.at[pl.ds(base, _CH)], di)
            pltpu.async_copy(g_hbm.at[si], gb, sem).wait()
            pltpu.async_copy(nxp_hbm.at[si], pb, sem).wait()
            pltpu.async_copy(qxp_hbm.at[di], qb, sem).wait()
            pltpu.sync_copy(gb, ge_hbm.at[pl.ds(base, _CH)])
            pltpu.sync_copy(pb, pxg_hbm.at[pl.ds(base, _CH)])
            pltpu.sync_copy(qb, qxg_hbm.at[pl.ds(base, _CH)])
            return carry

        lax.fori_loop(0, _NCH, body, 0)

    return k(g, nxp, qxp, esrc, edst)


def _edge_body(ge_ref, px_ref, qx_ref, w1_ref, b1_ref, w2_ref, b2_ref,
               wg_ref, wsh_ref, aaf_ref, lf_ref, sel_ref, selt_ref,
               t_ref, ex_ref):
    px = px_ref[...]
    qx = qx_ref[...]
    vec = px - qx
    length = jnp.sqrt(jnp.sum(vec * vec, axis=1, keepdims=True))
    inv = 1.0 / (length + 1e-9)
    x = vec[:, 0:1] * inv
    y = vec[:, 1:2] * inv
    z = vec[:, 2:3] * inv

    s3 = 3.0 ** 0.5
    s5 = 5.0 ** 0.5
    s15 = 15.0 ** 0.5
    wsh = wsh_ref[...]
    shW = (wsh[0:1, :]
           + (s3 * x) * wsh[1:2, :]
           + (s3 * y) * wsh[2:3, :]
           + (s3 * z) * wsh[3:4, :]
           + (s15 * x * y) * wsh[4:5, :]
           + (s15 * y * z) * wsh[5:6, :]
           + ((s5 / 2.0) * (3.0 * z * z - 1.0)) * wsh[6:7, :]
           + (s15 * x * z) * wsh[7:8, :]
           + ((s15 / 2.0) * (x * x - y * y)) * wsh[8:9, :])

    ang = length * lf_ref[...]
    lemb = jnp.concatenate([jnp.sin(ang), jnp.cos(ang)], axis=1)
    h = jnp.dot(lemb, w1_ref[...], preferred_element_type=jnp.float32) + b1_ref[...]
    h = h * jax.nn.sigmoid(h)
    es = jnp.dot(h, w2_ref[...], preferred_element_type=jnp.float32) + b2_ref[...]
    es = es * jax.nn.sigmoid(es)
    gate = jax.nn.sigmoid(
        jnp.dot(es, wg_ref[...], preferred_element_type=jnp.float32))

    env = 0.5 * (jnp.cos(jnp.pi * jnp.clip(length / _CUT, 0.0, 1.0)) + 1.0)

    m = ge_ref[...] * gate + shW
    la = jnp.where(m > 0, m, 0.2 * m) * aaf_ref[...]
    alpha = jnp.dot(la, sel_ref[...], preferred_element_type=jnp.float32)
    ex = jnp.exp(alpha) * env
    exw = jnp.dot(ex, selt_ref[...], preferred_element_type=jnp.float32)
    t_ref[...] = m * exw
    ex_ref[...] = jnp.concatenate([ex, jnp.zeros_like(ex)], axis=1)


def _edge_compute(ge, pxg, qxg, W1, b1, W2, b2, W_gate, W_sh, aaf, lf, sel, selt):
    blk = 2048
    return pl.pallas_call(
        _edge_body,
        grid=(_EP // blk,),
        in_specs=[
            pl.BlockSpec((blk, _D), lambda i: (i, 0)),
            pl.BlockSpec((blk, 16), lambda i: (i, 0)),
            pl.BlockSpec((blk, 16), lambda i: (i, 0)),
            pl.BlockSpec((_LE, _LE), lambda i: (0, 0)),
            pl.BlockSpec((_LE,), lambda i: (0,)),
            pl.BlockSpec((_LE, _LE), lambda i: (0, 0)),
            pl.BlockSpec((_LE,), lambda i: (0,)),
            pl.BlockSpec((_LE, _D), lambda i: (0, 0)),
            pl.BlockSpec((9, _D), lambda i: (0, 0)),
            pl.BlockSpec((1, _D), lambda i: (0, 0)),
            pl.BlockSpec((1, _LE // 2), lambda i: (0, 0)),
            pl.BlockSpec((_D, _H), lambda i: (0, 0)),
            pl.BlockSpec((_H, _D), lambda i: (0, 0)),
        ],
        out_specs=[
            pl.BlockSpec((blk, _D), lambda i: (i, 0)),
            pl.BlockSpec((blk, 16), lambda i: (i, 0)),
        ],
        out_shape=[
            jax.ShapeDtypeStruct((_EP, _D), jnp.float32),
            jax.ShapeDtypeStruct((_EP, 16), jnp.float32),
        ],
    )(ge, pxg, qxg, W1, b1, W2, b2, W_gate, W_sh, aaf, lf, sel, selt)


def _scatter_pass(p, t, exq, edst, z128, z16):
    lo = p * _PROWS

    @functools.partial(
        pl.kernel,
        mesh=_mesh(),
        out_type=[
            jax.ShapeDtypeStruct((_PROWS, _D), jnp.float32),
            jax.ShapeDtypeStruct((_PROWS, 16), jnp.float32),
        ],
        scratch_types=[
            pltpu.VMEM((_CH,), jnp.int32),
            pltpu.VMEM((_CH, 128), jnp.float32),
            pltpu.VMEM((_CH, 16), jnp.float32),
            pltpu.VMEM((64, 128), jnp.float32),
            pltpu.VMEM((64, 16), jnp.float32),
            pltpu.VMEM((_WR, 128), jnp.float32),
            pltpu.VMEM((_WR, 16), jnp.float32),
            pltpu.VMEM_SHARED((_PSR, 128), jnp.float32),
            pltpu.VMEM_SHARED((_PSR, 16), jnp.float32),
            pltpu.SemaphoreType.DMA,
        ],
        compiler_params=pltpu.CompilerParams(use_tc_tiling_on_sc=False, needs_layout_passes=False),
    )
    def k(t_hbm, ex_hbm, ed_hbm, z128_hbm, z16_hbm,
          s_hbm, den_hbm,
          idx, tb, eb, zb, dz, ob, dob, Ssh, Dsh, sem):
        c = lax.axis_index("c")
        s = lax.axis_index("s")
        col = c * 128
        nch = _EP // 16 // _CH

        pltpu.sync_copy(z128_hbm, zb)
        pltpu.sync_copy(z16_hbm, dz)
        for r in range(_PSR // 16 // 64):
            pltpu.sync_copy(zb, Ssh.at[pl.ds(s * (_PSR // 16) + r * 64, 64)])
            pltpu.sync_copy(dz, Dsh.at[pl.ds(s * (_PSR // 16) + r * 64, 64)])
        plsc.subcore_barrier()

        def body(j, carry):
            base = s * (_EP // 16) + j * _CH
            pltpu.sync_copy(ed_hbm.at[pl.ds(base, _CH)], idx)
            cmin = jnp.min(idx[pl.ds(0, 16)])
            cmax = jnp.max(idx[pl.ds(_CH - 16, 16)])
            overlap = jnp.logical_and(cmax >= lo, cmin < lo + _PROWS)

            @pl.when(overlap)
            def _():
                for grp in range(_CH // 16):
                    v = idx[pl.ds(grp * 16, 16)] - lo
                    inb = jnp.logical_and(v >= 0, v < _PROWS)
                    idx[pl.ds(grp * 16, 16)] = jnp.where(inb, v, _PROWS)

                pltpu.sync_copy(
                    t_hbm.at[pl.ds(base, _CH), pl.ds(col, 128)], tb)
                pltpu.sync_copy(tb, Ssh.at[idx], add=True)

                @pl.when(c == 0)
                def _():
                    pltpu.sync_copy(ex_hbm.at[pl.ds(base, _CH)], eb)
                    pltpu.sync_copy(eb, Dsh.at[idx], add=True)

            return carry

        lax.fori_loop(0, nch, body, 0)
        plsc.subcore_barrier()

        for r in range(3):
            cid = r * 16 + s

            @pl.when(cid < _PROWS // _WR)
            def _():
                rbase = cid * _WR
                pltpu.sync_copy(Ssh.at[pl.ds(rbase, _WR)], ob)
                pltpu.sync_copy(
                    ob, s_hbm.at[pl.ds(rbase, _WR), pl.ds(col, 128)])

                @pl.when(c == 0)
                def _():
                    pltpu.sync_copy(Dsh.at[pl.ds(rbase, _WR)], dob)
                    pltpu.sync_copy(dob, den_hbm.at[pl.ds(rbase, _WR)])

    return k(t, exq, edst, z128, z16)


def _scatter(t, exq, edst, z128, z16):
    parts = [_scatter_pass(p, t, exq, edst, z128, z16) for p in range(_PR)]
    S = jnp.concatenate([pp[0] for pp in parts], axis=0)
    den = jnp.concatenate([pp[1] for pp in parts], axis=0)
    return S, den


def _out_body(s_ref, den_ref, sel16_ref, wo_ref, o_ref):
    denrep = jnp.dot(den_ref[...], sel16_ref[...],
                     preferred_element_type=jnp.float32)
    R = s_ref[...] / (denrep + 1e-9)
    o_ref[...] = jnp.dot(R, wo_ref[...], preferred_element_type=jnp.float32)


def _finalize(S, den, sel16, W_out):
    blk = 2000
    return pl.pallas_call(
        _out_body,
        grid=(_NQ // blk,),
        in_specs=[
            pl.BlockSpec((blk, _D), lambda i: (i, 0)),
            pl.BlockSpec((blk, 16), lambda i: (i, 0)),
            pl.BlockSpec((16, _D), lambda i: (0, 0)),
            pl.BlockSpec((_D, _D), lambda i: (0, 0)),
        ],
        out_specs=pl.BlockSpec((blk, _D), lambda i: (i, 0)),
        out_shape=jax.ShapeDtypeStruct((_NQ, _D), jnp.float32),
    )(S, den, sel16, W_out)


def kernel(query_x, node_x, node_f, W1, b1, W2, b2, W_src, b_src, W_sh,
           W_gate, a_attn, W_out, edge_src, edge_dst):
    f32 = jnp.float32
    nxp = jnp.pad(node_x, ((0, 0), (0, 13)))
    qxp = jnp.pad(query_x, ((0, 0), (0, 13)))
    pad = _EP - _E
    esrc = jnp.concatenate([edge_src, jnp.zeros((pad,), jnp.int32)])
    edst = jnp.concatenate([edge_dst, jnp.full((pad,), _NQ, jnp.int32)])

    half = _LE // 2
    lf = jnp.exp(-jnp.log(10000.0) * jnp.arange(half, dtype=f32) / half)[None, :]
    sel = jnp.repeat(jnp.eye(_H, dtype=f32), _DH, axis=0)
    selt = sel.T
    sel16 = jnp.concatenate([selt, jnp.zeros((8, _D), f32)], axis=0)
    aaf = a_attn.reshape(1, _D)
    z128 = jnp.zeros((64, 128), f32)
    z16 = jnp.zeros((64, 16), f32)

    g = _node_transform(node_f, W_src, b_src)
    ge, pxg, qxg = _gather(g, nxp, qxp, esrc, edst)
    t, exq = _edge_compute(ge, pxg, qxg, W1, b1, W2, b2, W_gate, W_sh,
                           aaf, lf, sel, selt)
    S, den = _scatter(t, exq, edst, z128, z16)
    return _finalize(S, den, sel16, W_out)


# speedup vs baseline: 9.6603x; 1.0090x over previous
import functools

import jax
import jax.numpy as jnp
from jax import lax
from jax.experimental import pallas as pl
from jax.experimental.pallas import tpu as pltpu
from jax.experimental.pallas import tpu_sc as plsc

_NQ = 10000
_NP = 10000
_E = 160000
_D = 256
_H = 8
_DH = 32
_LE = 64
_CUT = 10.0

_NW = 32
_CH = 128
_EP = 163840
_EW = _EP // _NW
_NCH = _EW // _CH
_PR = 2
_PROWS = _NQ // _PR
_PSR = 5120
_WR = 125

def _mesh():
    return plsc.VectorSubcoreMesh(core_axis_name="c", subcore_axis_name="s")


def _node_mm_body(x_ref, w_ref, b_ref, px_ref, o_ref):
    o_ref[:, 0:_D] = (
        jnp.dot(x_ref[...], w_ref[...], preferred_element_type=jnp.float32)
        + b_ref[...]
    )
    o_ref[:, _D:_D + 16] = px_ref[...]


def _node_transform(node_f, W_src, b_src, nxp):
    blk = 2000
    return pl.pallas_call(
        _node_mm_body,
        grid=(_NP // blk,),
        in_specs=[
            pl.BlockSpec((blk, _D), lambda i: (i, 0)),
            pl.BlockSpec((_D, _D), lambda i: (0, 0)),
            pl.BlockSpec((_D,), lambda i: (0,)),
            pl.BlockSpec((blk, 16), lambda i: (i, 0)),
        ],
        out_specs=pl.BlockSpec((blk, _D + 16), lambda i: (i, 0)),
        out_shape=jax.ShapeDtypeStruct((_NP, _D + 16), jnp.float32),
    )(node_f, W_src, b_src, nxp)


_GB = 2


def _gather(comb, qxp, esrc, edst):
    bs = _GB * _CH

    @functools.partial(
        pl.kernel,
        mesh=_mesh(),
        out_type=[
            jax.ShapeDtypeStruct((_EP, _D + 16), jnp.float32),
            jax.ShapeDtypeStruct((_EP, 16), jnp.float32),
        ],
        scratch_types=[
            pltpu.VMEM((bs,), jnp.int32),
            pltpu.VMEM((bs,), jnp.int32),
            pltpu.VMEM((bs, _D + 16), jnp.float32),
            pltpu.VMEM((bs, 16), jnp.float32),
            pltpu.SemaphoreType.DMA,
        ],
        compiler_params=pltpu.CompilerParams(use_tc_tiling_on_sc=False, needs_layout_passes=False),
    )
    def k(comb_hbm, qxp_hbm, esrc_hbm, edst_hbm,
          ge_hbm, qxg_hbm, si, di, gb, qb, sem):
        wid = lax.axis_index("s") * 2 + lax.axis_index("c")

        def body(j, carry):
            base = wid * _EW + j * bs
            pltpu.sync_copy(esrc_hbm.at[pl.ds(base, bs)], si)
            pltpu.sync_copy(edst_hbm.at[pl.ds(base, bs)], di)
            ds = []
            for b in range(_GB):
                o = b * _CH
                ds.append(pltpu.async_copy(
                    comb_hbm.at[si.at[pl.ds(o, _CH)]],
                    gb.at[pl.ds(o, _CH)], sem))
                ds.append(pltpu.async_copy(
                    qxp_hbm.at[di.at[pl.ds(o, _CH)]],
                    qb.at[pl.ds(o, _CH)], sem))
            for d in ds:
                d.wait()
            pltpu.sync_copy(gb, ge_hbm.at[pl.ds(base, bs)])
            pltpu.sync_copy(qb, qxg_hbm.at[pl.ds(base, bs)])
            return carry

        lax.fori_loop(0, _NCH // _GB, body, 0)

    return k(comb, qxp, esrc, edst)


def _edge_body(gp_ref, qx_ref, w1_ref, b1_ref, w2_ref, b2_ref,
               wg_ref, wsh_ref, aaf_ref, lf_ref, sel_ref, selt_ref,
               tm_ref):
    gp = gp_ref[...]
    ge = gp[:, 0:_D]
    px = gp[:, _D:_D + 16]
    qx = qx_ref[...]
    vec = px - qx
    length = jnp.sqrt(jnp.sum(vec * vec, axis=1, keepdims=True))
    inv = 1.0 / (length + 1e-9)
    x = vec[:, 0:1] * inv
    y = vec[:, 1:2] * inv
    z = vec[:, 2:3] * inv

    s3 = 3.0 ** 0.5
    s5 = 5.0 ** 0.5
    s15 = 15.0 ** 0.5
    wsh = wsh_ref[...]
    shW = (wsh[0:1, :]
           + (s3 * x) * wsh[1:2, :]
           + (s3 * y) * wsh[2:3, :]
           + (s3 * z) * wsh[3:4, :]
           + (s15 * x * y) * wsh[4:5, :]
           + (s15 * y * z) * wsh[5:6, :]
           + ((s5 / 2.0) * (3.0 * z * z - 1.0)) * wsh[6:7, :]
           + (s15 * x * z) * wsh[7:8, :]
           + ((s15 / 2.0) * (x * x - y * y)) * wsh[8:9, :])

    ang = length * lf_ref[...]
    lemb = jnp.concatenate([jnp.sin(ang), jnp.cos(ang)], axis=1)
    h = jnp.dot(lemb, w1_ref[...], preferred_element_type=jnp.float32) + b1_ref[...]
    h = h * jax.nn.sigmoid(h)
    es = jnp.dot(h, w2_ref[...], preferred_element_type=jnp.float32) + b2_ref[...]
    es = es * jax.nn.sigmoid(es)
    gate = jax.nn.sigmoid(
        jnp.dot(es, wg_ref[...], preferred_element_type=jnp.float32))

    env = 0.5 * (jnp.cos(jnp.pi * jnp.clip(length / _CUT, 0.0, 1.0)) + 1.0)

    m = ge * gate + shW
    la = jnp.where(m > 0, m, 0.2 * m) * aaf_ref[...]
    alpha = jnp.dot(la, sel_ref[...], preferred_element_type=jnp.float32)
    ex = jnp.exp(alpha) * env
    exw = jnp.dot(ex, selt_ref[...], preferred_element_type=jnp.float32)
    t = m * exw
    z8 = jnp.zeros_like(ex)
    tm_ref[...] = jnp.concatenate(
        [t[:, 0:128], ex, z8, t[:, 128:256], z8, z8], axis=1)


def _edge_compute(geP, qxg, W1, b1, W2, b2, W_gate, W_sh, aaf, lf, sel, selt):
    blk = 2048
    return pl.pallas_call(
        _edge_body,
        grid=(_EP // blk,),
        in_specs=[
            pl.BlockSpec((blk, _D + 16), lambda i: (i, 0)),
            pl.BlockSpec((blk, 16), lambda i: (i, 0)),
            pl.BlockSpec((_LE, _LE), lambda i: (0, 0)),
            pl.BlockSpec((_LE,), lambda i: (0,)),
            pl.BlockSpec((_LE, _LE), lambda i: (0, 0)),
            pl.BlockSpec((_LE,), lambda i: (0,)),
            pl.BlockSpec((_LE, _D), lambda i: (0, 0)),
            pl.BlockSpec((9, _D), lambda i: (0, 0)),
            pl.BlockSpec((1, _D), lambda i: (0, 0)),
            pl.BlockSpec((1, _LE // 2), lambda i: (0, 0)),
            pl.BlockSpec((_D, _H), lambda i: (0, 0)),
            pl.BlockSpec((_H, _D), lambda i: (0, 0)),
        ],
        out_specs=pl.BlockSpec((blk, 288), lambda i: (i, 0)),
        out_shape=jax.ShapeDtypeStruct((_EP, 288), jnp.float32),
    )(geP, qxg, W1, b1, W2, b2, W_gate, W_sh, aaf, lf, sel, selt)


def _scatter(tm, edst, z144):
    @functools.partial(
        pl.kernel,
        mesh=_mesh(),
        out_type=[
            jax.ShapeDtypeStruct((_NQ, _D), jnp.float32),
            jax.ShapeDtypeStruct((_NQ, 16), jnp.float32),
        ],
        scratch_types=[
            pltpu.VMEM((_CH,), jnp.int32),
            pltpu.VMEM((_CH, 144), jnp.float32),
            pltpu.VMEM((64, 144), jnp.float32),
            pltpu.VMEM((_WR, 128), jnp.float32),
            pltpu.VMEM((_WR, 16), jnp.float32),
            pltpu.VMEM_SHARED((_PSR, 144), jnp.float32),
            pltpu.SemaphoreType.DMA,
        ],
        compiler_params=pltpu.CompilerParams(use_tc_tiling_on_sc=False, needs_layout_passes=False),
    )
    def k(tm_hbm, ed_hbm, z_hbm, s_hbm, den_hbm,
          idx, tb, zb, ob, dob, Ssh, sem):
        c = lax.axis_index("c")
        s = lax.axis_index("s")
        col = c * 144
        ocol = c * 128
        nch = _EP // 16 // _CH
        pltpu.sync_copy(z_hbm, zb)

        for p in range(_PR):
            lo = p * _PROWS
            for r in range(_PSR // 16 // 64):
                pltpu.sync_copy(
                    zb, Ssh.at[pl.ds(s * (_PSR // 16) + r * 64, 64)])
            plsc.subcore_barrier()

            def body(j, carry):
                base = s * (_EP // 16) + j * _CH
                pltpu.sync_copy(ed_hbm.at[pl.ds(base, _CH)], idx)
                cmin = jnp.min(idx[pl.ds(0, 16)])
                cmax = jnp.max(idx[pl.ds(_CH - 16, 16)])
                overlap = jnp.logical_and(cmax >= lo, cmin < lo + _PROWS)

                @pl.when(overlap)
                def _():
                    for grp in range(_CH // 16):
                        v = idx[pl.ds(grp * 16, 16)] - lo
                        inb = jnp.logical_and(v >= 0, v < _PROWS)
                        idx[pl.ds(grp * 16, 16)] = jnp.where(inb, v, _PROWS)

                    pltpu.sync_copy(
                        tm_hbm.at[pl.ds(base, _CH), pl.ds(col, 144)], tb)
                    pltpu.sync_copy(tb, Ssh.at[idx], add=True)

                return carry

            lax.fori_loop(0, nch, body, 0)
            plsc.subcore_barrier()

            for r in range(3):
                cid = r * 16 + s

                @pl.when(cid < _PROWS // _WR)
                def _():
                    rbase = cid * _WR
                    pltpu.sync_copy(
                        Ssh.at[pl.ds(rbase, _WR), pl.ds(0, 128)], ob)
                    pltpu.sync_copy(
                        ob, s_hbm.at[pl.ds(lo + rbase, _WR),
                                     pl.ds(ocol, 128)])

                    @pl.when(c == 0)
                    def _():
                        pltpu.sync_copy(
                            Ssh.at[pl.ds(rbase, _WR), pl.ds(128, 16)], dob)
                        pltpu.sync_copy(
                            dob, den_hbm.at[pl.ds(lo + rbase, _WR)])
            plsc.subcore_barrier()

    return k(tm, edst, z144)


def _out_body(s_ref, den_ref, sel16_ref, wo_ref, o_ref):
    denrep = jnp.dot(den_ref[...], sel16_ref[...],
                     preferred_element_type=jnp.float32)
    R = s_ref[...] / (denrep + 1e-9)
    o_ref[...] = jnp.dot(R, wo_ref[...], preferred_element_type=jnp.float32)


def _finalize(S, den, sel16, W_out):
    blk = 2000
    return pl.pallas_call(
        _out_body,
        grid=(_NQ // blk,),
        in_specs=[
            pl.BlockSpec((blk, _D), lambda i: (i, 0)),
            pl.BlockSpec((blk, 16), lambda i: (i, 0)),
            pl.BlockSpec((16, _D), lambda i: (0, 0)),
            pl.BlockSpec((_D, _D), lambda i: (0, 0)),
        ],
        out_specs=pl.BlockSpec((blk, _D), lambda i: (i, 0)),
        out_shape=jax.ShapeDtypeStruct((_NQ, _D), jnp.float32),
    )(S, den, sel16, W_out)


def kernel(query_x, node_x, node_f, W1, b1, W2, b2, W_src, b_src, W_sh,
           W_gate, a_attn, W_out, edge_src, edge_dst):
    f32 = jnp.float32
    nxp = jnp.pad(node_x, ((0, 0), (0, 13)))
    qxp = jnp.pad(query_x, ((0, 0), (0, 13)))
    pad = _EP - _E
    esrc = jnp.concatenate([edge_src, jnp.zeros((pad,), jnp.int32)])
    edst = jnp.concatenate([edge_dst, jnp.full((pad,), _NQ, jnp.int32)])

    half = _LE // 2
    lf = jnp.exp(-jnp.log(10000.0) * jnp.arange(half, dtype=f32) / half)[None, :]
    sel = jnp.repeat(jnp.eye(_H, dtype=f32), _DH, axis=0)
    selt = sel.T
    sel16 = jnp.concatenate([selt, jnp.zeros((8, _D), f32)], axis=0)
    aaf = a_attn.reshape(1, _D)
    z144 = jnp.zeros((64, 144), f32)

    comb = _node_transform(node_f, W_src, b_src, nxp)
    geP, qxg = _gather(comb, qxp, esrc, edst)
    tm = _edge_compute(geP, qxg, W1, b1, W2, b2, W_gate, W_sh,
                       aaf, lf, sel, selt)
    S, den = _scatter(tm, edst, z144)
    return _finalize(S, den, sel16, W_out)


# speedup vs baseline: 10.8448x; 1.1226x over previous
import functools

import jax
import jax.numpy as jnp
from jax import lax
from jax.experimental import pallas as pl
from jax.experimental.pallas import tpu as pltpu
from jax.experimental.pallas import tpu_sc as plsc

_NQ = 10000
_NP = 10000
_E = 160000
_D = 256
_H = 8
_DH = 32
_LE = 64
_CUT = 10.0

_NW = 32
_CH = 128
_EP = 163840
_EW = _EP // _NW
_NCH = _EW // _CH
_PR = 2
_PROWS = _NQ // _PR
_PSR = 5120
_WR = 125

def _mesh():
    return plsc.VectorSubcoreMesh(core_axis_name="c", subcore_axis_name="s")


def _node_mm_body(x_ref, w_ref, b_ref, px_ref, o_ref):
    o_ref[:, 0:_D] = (
        jnp.dot(x_ref[...], w_ref[...], preferred_element_type=jnp.float32)
        + b_ref[...]
    )
    o_ref[:, _D:_D + 16] = px_ref[...]


def _node_transform(node_f, W_src, b_src, nxp):
    blk = 2000
    return pl.pallas_call(
        _node_mm_body,
        grid=(_NP // blk,),
        in_specs=[
            pl.BlockSpec((blk, _D), lambda i: (i, 0)),
            pl.BlockSpec((_D, _D), lambda i: (0, 0)),
            pl.BlockSpec((_D,), lambda i: (0,)),
            pl.BlockSpec((blk, 16), lambda i: (i, 0)),
        ],
        out_specs=pl.BlockSpec((blk, _D + 16), lambda i: (i, 0)),
        out_shape=jax.ShapeDtypeStruct((_NP, _D + 16), jnp.float32),
    )(node_f, W_src, b_src, nxp)


_GB = 2


def _gather(comb, qxp, esrc, edst):
    bs = _GB * _CH

    @functools.partial(
        pl.kernel,
        mesh=_mesh(),
        out_type=[
            jax.ShapeDtypeStruct((_EP, _D + 16), jnp.float32),
            jax.ShapeDtypeStruct((_EP, 16), jnp.float32),
        ],
        scratch_types=[
            pltpu.VMEM((bs,), jnp.int32),
            pltpu.VMEM((bs,), jnp.int32),
            pltpu.VMEM((bs, _D + 16), jnp.float32),
            pltpu.VMEM((bs, 16), jnp.float32),
            pltpu.SemaphoreType.DMA,
        ],
        compiler_params=pltpu.CompilerParams(use_tc_tiling_on_sc=False, needs_layout_passes=False),
    )
    def k(comb_hbm, qxp_hbm, esrc_hbm, edst_hbm,
          ge_hbm, qxg_hbm, si, di, gb, qb, sem):
        wid = lax.axis_index("s") * 2 + lax.axis_index("c")

        def body(j, carry):
            base = wid * _EW + j * bs
            pltpu.sync_copy(esrc_hbm.at[pl.ds(base, bs)], si)
            pltpu.sync_copy(edst_hbm.at[pl.ds(base, bs)], di)
            ds = []
            for b in range(_GB):
                o = b * _CH
                ds.append(pltpu.async_copy(
                    comb_hbm.at[si.at[pl.ds(o, _CH)]],
                    gb.at[pl.ds(o, _CH)], sem))
                ds.append(pltpu.async_copy(
                    qxp_hbm.at[di.at[pl.ds(o, _CH)]],
                    qb.at[pl.ds(o, _CH)], sem))
            for d in ds:
                d.wait()
            pltpu.sync_copy(gb, ge_hbm.at[pl.ds(base, bs)])
            pltpu.sync_copy(qb, qxg_hbm.at[pl.ds(base, bs)])
            return carry

        lax.fori_loop(0, _NCH // _GB, body, 0)

    return k(comb, qxp, esrc, edst)


def _edge_body(gp_ref, qx_ref, w1_ref, b1_ref, w2_ref, b2_ref,
               wg_ref, wsh_ref, aaf_ref, lf_ref, sel_ref, selt_ref,
               tm_ref):
    gp = gp_ref[...]
    ge = gp[:, 0:_D]
    px = gp[:, _D:_D + 16]
    qx = qx_ref[...]
    vec = px - qx
    length = jnp.sqrt(jnp.sum(vec * vec, axis=1, keepdims=True))
    inv = 1.0 / (length + 1e-9)
    x = vec[:, 0:1] * inv
    y = vec[:, 1:2] * inv
    z = vec[:, 2:3] * inv

    s3 = 3.0 ** 0.5
    s5 = 5.0 ** 0.5
    s15 = 15.0 ** 0.5
    one = jnp.ones_like(x)
    shvec = jnp.concatenate(
        [one, s3 * x, s3 * y, s3 * z,
         s15 * x * y, s15 * y * z,
         (s5 / 2.0) * (3.0 * z * z - 1.0),
         s15 * x * z, (s15 / 2.0) * (x * x - y * y),
         jnp.zeros((x.shape[0], 7), x.dtype)], axis=1)
    shW = jnp.dot(shvec, wsh_ref[...], preferred_element_type=jnp.float32)

    ang = length * lf_ref[...]
    hp = jnp.pi / 2.0
    clipped = jnp.clip(length * (1.0 / _CUT), 0.0, 1.0)
    sing = jnp.concatenate(
        [ang, ang + hp, jnp.pi * clipped + hp], axis=1)
    sv = jnp.sin(sing)
    lemb = sv[:, 0:_LE]
    env = 0.5 * (sv[:, _LE:_LE + 1] + 1.0)

    h = jnp.dot(lemb, w1_ref[...], preferred_element_type=jnp.float32) + b1_ref[...]
    h = h / (1.0 + jnp.exp(-h))
    es = jnp.dot(h, w2_ref[...], preferred_element_type=jnp.float32) + b2_ref[...]
    es = es / (1.0 + jnp.exp(-es))
    u = jnp.dot(es, wg_ref[...], preferred_element_type=jnp.float32)
    gate = 1.0 / (1.0 + jnp.exp(-u))

    m = ge * gate + shW
    la = jnp.maximum(m, 0.2 * m) * aaf_ref[...]
    alpha = jnp.dot(la, sel_ref[...], preferred_element_type=jnp.float32)
    ex = jnp.exp(alpha) * env
    exw = jnp.dot(ex, selt_ref[...], preferred_element_type=jnp.float32)
    t = m * exw
    z8 = jnp.zeros_like(ex)
    tm_ref[...] = jnp.concatenate(
        [t[:, 0:128], ex, z8, t[:, 128:256], z8, z8], axis=1)


def _edge_compute(geP, qxg, W1, b1, W2, b2, W_gate, W_sh, aaf, lf, sel, selt):
    blk = 2048
    return pl.pallas_call(
        _edge_body,
        grid=(_EP // blk,),
        in_specs=[
            pl.BlockSpec((blk, _D + 16), lambda i: (i, 0)),
            pl.BlockSpec((blk, 16), lambda i: (i, 0)),
            pl.BlockSpec((_LE, _LE), lambda i: (0, 0)),
            pl.BlockSpec((_LE,), lambda i: (0,)),
            pl.BlockSpec((_LE, _LE), lambda i: (0, 0)),
            pl.BlockSpec((_LE,), lambda i: (0,)),
            pl.BlockSpec((_LE, _D), lambda i: (0, 0)),
            pl.BlockSpec((16, _D), lambda i: (0, 0)),
            pl.BlockSpec((1, _D), lambda i: (0, 0)),
            pl.BlockSpec((1, _LE // 2), lambda i: (0, 0)),
            pl.BlockSpec((_D, _H), lambda i: (0, 0)),
            pl.BlockSpec((_H, _D), lambda i: (0, 0)),
        ],
        out_specs=pl.BlockSpec((blk, 288), lambda i: (i, 0)),
        out_shape=jax.ShapeDtypeStruct((_EP, 288), jnp.float32),
    )(geP, qxg, W1, b1, W2, b2, W_gate, W_sh, aaf, lf, sel, selt)


def _scatter(tm, edst, z144):
    @functools.partial(
        pl.kernel,
        mesh=_mesh(),
        out_type=[
            jax.ShapeDtypeStruct((_NQ, _D), jnp.float32),
            jax.ShapeDtypeStruct((_NQ, 16), jnp.float32),
        ],
        scratch_types=[
            pltpu.VMEM((_CH,), jnp.int32),
            pltpu.VMEM((_CH, 144), jnp.float32),
            pltpu.VMEM((64, 144), jnp.float32),
            pltpu.VMEM((_WR, 128), jnp.float32),
            pltpu.VMEM((_WR, 16), jnp.float32),
            pltpu.VMEM_SHARED((_PSR, 144), jnp.float32),
            pltpu.SemaphoreType.DMA,
        ],
        compiler_params=pltpu.CompilerParams(use_tc_tiling_on_sc=False, needs_layout_passes=False),
    )
    def k(tm_hbm, ed_hbm, z_hbm, s_hbm, den_hbm,
          idx, tb, zb, ob, dob, Ssh, sem):
        c = lax.axis_index("c")
        s = lax.axis_index("s")
        col = c * 144
        ocol = c * 128
        nch = _EP // 16 // _CH
        pltpu.sync_copy(z_hbm, zb)

        for p in range(_PR):
            lo = p * _PROWS
            for r in range(_PSR // 16 // 64):
                pltpu.sync_copy(
                    zb, Ssh.at[pl.ds(s * (_PSR // 16) + r * 64, 64)])
            plsc.subcore_barrier()

            def body(j, carry):
                base = s * (_EP // 16) + j * _CH
                pltpu.sync_copy(ed_hbm.at[pl.ds(base, _CH)], idx)
                cmin = jnp.min(idx[pl.ds(0, 16)])
                cmax = jnp.max(idx[pl.ds(_CH - 16, 16)])
                overlap = jnp.logical_and(cmax >= lo, cmin < lo + _PROWS)

                @pl.when(overlap)
                def _():
                    for grp in range(_CH // 16):
                        v = idx[pl.ds(grp * 16, 16)] - lo
                        inb = jnp.logical_and(v >= 0, v < _PROWS)
                        idx[pl.ds(grp * 16, 16)] = jnp.where(inb, v, _PROWS)

                    pltpu.sync_copy(
                        tm_hbm.at[pl.ds(base, _CH), pl.ds(col, 144)], tb)
                    pltpu.sync_copy(tb, Ssh.at[idx], add=True)

                return carry

            lax.fori_loop(0, nch, body, 0)
            plsc.subcore_barrier()

            for r in range(3):
                cid = r * 16 + s

                @pl.when(cid < _PROWS // _WR)
                def _():
                    rbase = cid * _WR
                    pltpu.sync_copy(
                        Ssh.at[pl.ds(rbase, _WR), pl.ds(0, 128)], ob)
                    pltpu.sync_copy(
                        ob, s_hbm.at[pl.ds(lo + rbase, _WR),
                                     pl.ds(ocol, 128)])

                    @pl.when(c == 0)
                    def _():
                        pltpu.sync_copy(
                            Ssh.at[pl.ds(rbase, _WR), pl.ds(128, 16)], dob)
                        pltpu.sync_copy(
                            dob, den_hbm.at[pl.ds(lo + rbase, _WR)])
            plsc.subcore_barrier()

    return k(tm, edst, z144)


def _out_body(s_ref, den_ref, sel16_ref, wo_ref, o_ref):
    denrep = jnp.dot(den_ref[...], sel16_ref[...],
                     preferred_element_type=jnp.float32)
    R = s_ref[...] / (denrep + 1e-9)
    o_ref[...] = jnp.dot(R, wo_ref[...], preferred_element_type=jnp.float32)


def _finalize(S, den, sel16, W_out):
    blk = 2000
    return pl.pallas_call(
        _out_body,
        grid=(_NQ // blk,),
        in_specs=[
            pl.BlockSpec((blk, _D), lambda i: (i, 0)),
            pl.BlockSpec((blk, 16), lambda i: (i, 0)),
            pl.BlockSpec((16, _D), lambda i: (0, 0)),
            pl.BlockSpec((_D, _D), lambda i: (0, 0)),
        ],
        out_specs=pl.BlockSpec((blk, _D), lambda i: (i, 0)),
        out_shape=jax.ShapeDtypeStruct((_NQ, _D), jnp.float32),
    )(S, den, sel16, W_out)


def kernel(query_x, node_x, node_f, W1, b1, W2, b2, W_src, b_src, W_sh,
           W_gate, a_attn, W_out, edge_src, edge_dst):
    f32 = jnp.float32
    nxp = jnp.pad(node_x, ((0, 0), (0, 13)))
    qxp = jnp.pad(query_x, ((0, 0), (0, 13)))
    pad = _EP - _E
    esrc = jnp.concatenate([edge_src, jnp.zeros((pad,), jnp.int32)])
    edst = jnp.concatenate([edge_dst, jnp.full((pad,), _NQ, jnp.int32)])

    half = _LE // 2
    lf = jnp.exp(-jnp.log(10000.0) * jnp.arange(half, dtype=f32) / half)[None, :]
    sel = jnp.repeat(jnp.eye(_H, dtype=f32), _DH, axis=0)
    selt = sel.T
    sel16 = jnp.concatenate([selt, jnp.zeros((8, _D), f32)], axis=0)
    aaf = a_attn.reshape(1, _D)
    wsh16 = jnp.pad(W_sh, ((0, 7), (0, 0)))
    z144 = jnp.zeros((64, 144), f32)

    comb = _node_transform(node_f, W_src, b_src, nxp)
    geP, qxg = _gather(comb, qxp, esrc, edst)
    tm = _edge_compute(geP, qxg, W1, b1, W2, b2, W_gate, wsh16,
                       aaf, lf, sel, selt)
    S, den = _scatter(tm, edst, z144)
    return _finalize(S, den, sel16, W_out)


# speedup vs baseline: 11.0974x; 1.0233x over previous
import functools

import jax
import jax.numpy as jnp
from jax import lax
from jax.experimental import pallas as pl
from jax.experimental.pallas import tpu as pltpu
from jax.experimental.pallas import tpu_sc as plsc

_NQ = 10000
_NP = 10000
_E = 160000
_D = 256
_H = 8
_DH = 32
_LE = 64
_CUT = 10.0

_NW = 32
_CH = 128
_EP = 163840
_EW = _EP // _NW
_NCH = _EW // _CH
_PR = 2
_PROWS = _NQ // _PR
_PSR = 5120
_WR = 125

def _mesh():
    return plsc.VectorSubcoreMesh(core_axis_name="c", subcore_axis_name="s")


def _node_mm_body(x_ref, w_ref, b_ref, px_ref, o_ref):
    o_ref[:, 0:_D] = (
        jnp.dot(x_ref[...], w_ref[...], preferred_element_type=jnp.float32)
        + b_ref[...]
    )
    o_ref[:, _D:_D + 16] = px_ref[...]


def _node_transform(node_f, W_src, b_src, nxp):
    blk = 2000
    return pl.pallas_call(
        _node_mm_body,
        grid=(_NP // blk,),
        in_specs=[
            pl.BlockSpec((blk, _D), lambda i: (i, 0)),
            pl.BlockSpec((_D, _D), lambda i: (0, 0)),
            pl.BlockSpec((_D,), lambda i: (0,)),
            pl.BlockSpec((blk, 16), lambda i: (i, 0)),
        ],
        out_specs=pl.BlockSpec((blk, _D + 16), lambda i: (i, 0)),
        out_shape=jax.ShapeDtypeStruct((_NP, _D + 16), jnp.float32),
    )(node_f, W_src, b_src, nxp)


def _gather(comb, qxp, esrc, edst):
    @functools.partial(
        pl.kernel,
        mesh=_mesh(),
        out_type=[
            jax.ShapeDtypeStruct((_EP, _D + 16), jnp.float32),
            jax.ShapeDtypeStruct((_EP, 16), jnp.float32),
        ],
        scratch_types=[
            pltpu.VMEM((2, _CH), jnp.int32),
            pltpu.VMEM((2, _CH), jnp.int32),
            pltpu.VMEM((2, _CH, _D + 16), jnp.float32),
            pltpu.VMEM((2, _CH, 16), jnp.float32),
            pltpu.SemaphoreType.DMA,
            pltpu.SemaphoreType.DMA,
            pltpu.SemaphoreType.DMA,
            pltpu.SemaphoreType.DMA,
        ],
        compiler_params=pltpu.CompilerParams(use_tc_tiling_on_sc=False, needs_layout_passes=False),
    )
    def k(comb_hbm, qxp_hbm, esrc_hbm, edst_hbm,
          ge_hbm, qxg_hbm, si, di, gb, qb, g0, g1, w0, w1):
        wid = lax.axis_index("s") * 2 + lax.axis_index("c")
        gsem = (g0, g1)
        wsem = (w0, w1)

        def body(i, carry):
            for b in range(2):
                j = i * 2 + b
                base = wid * _EW + j * _CH

                @pl.when(i > 0)
                def _():
                    pb = wid * _EW + (j - 2) * _CH
                    pltpu.make_async_copy(
                        gb.at[b], ge_hbm.at[pl.ds(pb, _CH)], wsem[b]).wait()
                    pltpu.make_async_copy(
                        qb.at[b], qxg_hbm.at[pl.ds(pb, _CH)], wsem[b]).wait()

                pltpu.sync_copy(esrc_hbm.at[pl.ds(base, _CH)], si.at[b])
                pltpu.sync_copy(edst_hbm.at[pl.ds(base, _CH)], di.at[b])
                pltpu.async_copy(comb_hbm.at[si.at[b]], gb.at[b], gsem[b])
                pltpu.async_copy(qxp_hbm.at[di.at[b]], qb.at[b], gsem[b])

            for b in range(2):
                j = i * 2 + b
                base = wid * _EW + j * _CH
                pltpu.make_async_copy(
                    comb_hbm.at[si.at[b]], gb.at[b], gsem[b]).wait()
                pltpu.make_async_copy(
                    qxp_hbm.at[di.at[b]], qb.at[b], gsem[b]).wait()
                pltpu.async_copy(gb.at[b], ge_hbm.at[pl.ds(base, _CH)],
                                 wsem[b])
                pltpu.async_copy(qb.at[b], qxg_hbm.at[pl.ds(base, _CH)],
                                 wsem[b])
            return carry

        nit = _NCH // 2
        lax.fori_loop(0, nit, body, 0)
        for b in range(2):
            last = wid * _EW + ((nit - 1) * 2 + b) * _CH
            pltpu.make_async_copy(
                gb.at[b], ge_hbm.at[pl.ds(last, _CH)], wsem[b]).wait()
            pltpu.make_async_copy(
                qb.at[b], qxg_hbm.at[pl.ds(last, _CH)], wsem[b]).wait()

    return k(comb, qxp, esrc, edst)


def _edge_body(gp_ref, qx_ref, w1_ref, b1_ref, w2_ref, b2_ref,
               wg_ref, wsh_ref, aaf_ref, lf_ref, sel_ref, selt_ref,
               tm_ref):
    gp = gp_ref[...]
    ge = gp[:, 0:_D]
    px = gp[:, _D:_D + 16]
    qx = qx_ref[...]
    vec = px - qx
    length = jnp.sqrt(jnp.sum(vec * vec, axis=1, keepdims=True))
    inv = 1.0 / (length + 1e-9)
    x = vec[:, 0:1] * inv
    y = vec[:, 1:2] * inv
    z = vec[:, 2:3] * inv

    s3 = 3.0 ** 0.5
    s5 = 5.0 ** 0.5
    s15 = 15.0 ** 0.5
    one = jnp.ones_like(x)
    shvec = jnp.concatenate(
        [one, s3 * x, s3 * y, s3 * z,
         s15 * x * y, s15 * y * z,
         (s5 / 2.0) * (3.0 * z * z - 1.0),
         s15 * x * z, (s15 / 2.0) * (x * x - y * y),
         jnp.zeros((x.shape[0], 7), x.dtype)], axis=1)
    shW = jnp.dot(shvec, wsh_ref[...], preferred_element_type=jnp.float32)

    ang = length * lf_ref[...]
    hp = jnp.pi / 2.0
    clipped = jnp.clip(length * (1.0 / _CUT), 0.0, 1.0)
    sing = jnp.concatenate(
        [ang, ang + hp, jnp.pi * clipped + hp], axis=1)
    sv = jnp.sin(sing)
    lemb = sv[:, 0:_LE]
    env = 0.5 * (sv[:, _LE:_LE + 1] + 1.0)

    h = jnp.dot(lemb, w1_ref[...], preferred_element_type=jnp.float32) + b1_ref[...]
    h = h / (1.0 + jnp.exp(-h))
    es = jnp.dot(h, w2_ref[...], preferred_element_type=jnp.float32) + b2_ref[...]
    es = es / (1.0 + jnp.exp(-es))
    u = jnp.dot(es, wg_ref[...], preferred_element_type=jnp.float32)
    gate = 1.0 / (1.0 + jnp.exp(-u))

    m = ge * gate + shW
    la = jnp.maximum(m, 0.2 * m) * aaf_ref[...]
    alpha = jnp.dot(la, sel_ref[...], preferred_element_type=jnp.float32)
    ex = jnp.exp(alpha) * env
    exw = jnp.dot(ex, selt_ref[...], preferred_element_type=jnp.float32)
    t = m * exw
    z8 = jnp.zeros_like(ex)
    tm_ref[...] = jnp.concatenate(
        [t[:, 0:128], ex, z8, t[:, 128:256], z8, z8], axis=1)


def _edge_compute(geP, qxg, W1, b1, W2, b2, W_gate, W_sh, aaf, lf, sel, selt):
    blk = 2048
    return pl.pallas_call(
        _edge_body,
        grid=(_EP // blk,),
        in_specs=[
            pl.BlockSpec((blk, _D + 16), lambda i: (i, 0)),
            pl.BlockSpec((blk, 16), lambda i: (i, 0)),
            pl.BlockSpec((_LE, _LE), lambda i: (0, 0)),
            pl.BlockSpec((_LE,), lambda i: (0,)),
            pl.BlockSpec((_LE, _LE), lambda i: (0, 0)),
            pl.BlockSpec((_LE,), lambda i: (0,)),
            pl.BlockSpec((_LE, _D), lambda i: (0, 0)),
            pl.BlockSpec((16, _D), lambda i: (0, 0)),
            pl.BlockSpec((1, _D), lambda i: (0, 0)),
            pl.BlockSpec((1, _LE // 2), lambda i: (0, 0)),
            pl.BlockSpec((_D, _H), lambda i: (0, 0)),
            pl.BlockSpec((_H, _D), lambda i: (0, 0)),
        ],
        out_specs=pl.BlockSpec((blk, 288), lambda i: (i, 0)),
        out_shape=jax.ShapeDtypeStruct((_EP, 288), jnp.float32),
    )(geP, qxg, W1, b1, W2, b2, W_gate, W_sh, aaf, lf, sel, selt)


def _scatter(tm, edst, z144):
    @functools.partial(
        pl.kernel,
        mesh=_mesh(),
        out_type=[
            jax.ShapeDtypeStruct((_NQ, _D), jnp.float32),
            jax.ShapeDtypeStruct((_NQ, 16), jnp.float32),
        ],
        scratch_types=[
            pltpu.VMEM((_CH,), jnp.int32),
            pltpu.VMEM((_CH, 144), jnp.float32),
            pltpu.VMEM((64, 144), jnp.float32),
            pltpu.VMEM((_WR, 128), jnp.float32),
            pltpu.VMEM((_WR, 16), jnp.float32),
            pltpu.VMEM_SHARED((_PSR, 144), jnp.float32),
            pltpu.SemaphoreType.DMA,
        ],
        compiler_params=pltpu.CompilerParams(use_tc_tiling_on_sc=False, needs_layout_passes=False),
    )
    def k(tm_hbm, ed_hbm, z_hbm, s_hbm, den_hbm,
          idx, tb, zb, ob, dob, Ssh, sem):
        c = lax.axis_index("c")
        s = lax.axis_index("s")
        col = c * 144
        ocol = c * 128
        nch = _EP // 16 // _CH
        pltpu.sync_copy(z_hbm, zb)

        for p in range(_PR):
            lo = p * _PROWS
            for r in range(_PSR // 16 // 64):
                pltpu.sync_copy(
                    zb, Ssh.at[pl.ds(s * (_PSR // 16) + r * 64, 64)])
            plsc.subcore_barrier()

            def body(j, carry):
                base = s * (_EP // 16) + j * _CH
                pltpu.sync_copy(ed_hbm.at[pl.ds(base, _CH)], idx)
                cmin = jnp.min(idx[pl.ds(0, 16)])
                cmax = jnp.max(idx[pl.ds(_CH - 16, 16)])
                overlap = jnp.logical_and(cmax >= lo, cmin < lo + _PROWS)

                @pl.when(overlap)
                def _():
                    for grp in range(_CH // 16):
                        v = idx[pl.ds(grp * 16, 16)] - lo
                        inb = jnp.logical_and(v >= 0, v < _PROWS)
                        idx[pl.ds(grp * 16, 16)] = jnp.where(inb, v, _PROWS)

                    pltpu.sync_copy(
                        tm_hbm.at[pl.ds(base, _CH), pl.ds(col, 144)], tb)
                    pltpu.sync_copy(tb, Ssh.at[idx], add=True)

                return carry

            lax.fori_loop(0, nch, body, 0)
            plsc.subcore_barrier()

            for r in range(3):
                cid = r * 16 + s

                @pl.when(cid < _PROWS // _WR)
                def _():
                    rbase = cid * _WR
                    pltpu.sync_copy(
                        Ssh.at[pl.ds(rbase, _WR), pl.ds(0, 128)], ob)
                    pltpu.sync_copy(
                        ob, s_hbm.at[pl.ds(lo + rbase, _WR),
                                     pl.ds(ocol, 128)])

                    @pl.when(c == 0)
                    def _():
                        pltpu.sync_copy(
                            Ssh.at[pl.ds(rbase, _WR), pl.ds(128, 16)], dob)
                        pltpu.sync_copy(
                            dob, den_hbm.at[pl.ds(lo + rbase, _WR)])
            plsc.subcore_barrier()

    return k(tm, edst, z144)


def _out_body(s_ref, den_ref, sel16_ref, wo_ref, o_ref):
    denrep = jnp.dot(den_ref[...], sel16_ref[...],
                     preferred_element_type=jnp.float32)
    R = s_ref[...] / (denrep + 1e-9)
    o_ref[...] = jnp.dot(R, wo_ref[...], preferred_element_type=jnp.float32)


def _finalize(S, den, sel16, W_out):
    blk = 2000
    return pl.pallas_call(
        _out_body,
        grid=(_NQ // blk,),
        in_specs=[
            pl.BlockSpec((blk, _D), lambda i: (i, 0)),
            pl.BlockSpec((blk, 16), lambda i: (i, 0)),
            pl.BlockSpec((16, _D), lambda i: (0, 0)),
            pl.BlockSpec((_D, _D), lambda i: (0, 0)),
        ],
        out_specs=pl.BlockSpec((blk, _D), lambda i: (i, 0)),
        out_shape=jax.ShapeDtypeStruct((_NQ, _D), jnp.float32),
    )(S, den, sel16, W_out)


def kernel(query_x, node_x, node_f, W1, b1, W2, b2, W_src, b_src, W_sh,
           W_gate, a_attn, W_out, edge_src, edge_dst):
    f32 = jnp.float32
    nxp = jnp.pad(node_x, ((0, 0), (0, 13)))
    qxp = jnp.pad(query_x, ((0, 0), (0, 13)))
    pad = _EP - _E
    esrc = jnp.concatenate([edge_src, jnp.zeros((pad,), jnp.int32)])
    edst = jnp.concatenate([edge_dst, jnp.full((pad,), _NQ, jnp.int32)])

    half = _LE // 2
    lf = jnp.exp(-jnp.log(10000.0) * jnp.arange(half, dtype=f32) / half)[None, :]
    sel = jnp.repeat(jnp.eye(_H, dtype=f32), _DH, axis=0)
    selt = sel.T
    sel16 = jnp.concatenate([selt, jnp.zeros((8, _D), f32)], axis=0)
    aaf = a_attn.reshape(1, _D)
    wsh16 = jnp.pad(W_sh, ((0, 7), (0, 0)))
    z144 = jnp.zeros((64, 144), f32)

    comb = _node_transform(node_f, W_src, b_src, nxp)
    geP, qxg = _gather(comb, qxp, esrc, edst)
    tm = _edge_compute(geP, qxg, W1, b1, W2, b2, W_gate, wsh16,
                       aaf, lf, sel, selt)
    S, den = _scatter(tm, edst, z144)
    return _finalize(S, den, sel16, W_out)
